# baseline (speedup 1.0000x reference)
import functools

import jax
import jax.numpy as jnp
from jax import lax
from jax.experimental import pallas as pl
from jax.experimental.pallas import tpu as pltpu
from jax.experimental.pallas import tpu_sc as plsc

B = 16384
NUM = 13
NCAT = 26
VOCAB = 100000
D = 32
NT = 1 + NUM + NCAT
NW = 32
NB = 64
ROWS_PER_W = B // NW
NCHUNKS = ROWS_PER_W // NB

FSC = 10
FTC = NCAT - FSC
VTILES = VOCAB // 128
VMAIN = VTILES * 128
A_ROWS = FSC * VOCAB

VCH = 4096
NBLK = (VOCAB + VCH - 1) // VCH
ROWS_PER_F = NBLK * VCH // 4


def _relay_body(tbl_hbm, aux_hbm, out_hbm, stg0, stg1, out_v, dsem0, dsem1):
    stgs = (stg0, stg1)
    wid = lax.axis_index("s") * 2 + lax.axis_index("c")
    iota16 = lax.iota(jnp.int32, 16)
    midxs = [mb * 16 + iota16 for mb in range(8)]
    sidxs = [m * 32 for m in midxs]
    sems = (dsem0, dsem1)

    @pl.when(wid == 0)
    def _():
        pltpu.sync_copy(aux_hbm, out_hbm.at[pl.ds(FSC * VMAIN * D,
                                                  FSC * (VOCAB - VMAIN) * D)])

    def _slab(f, k):
        vt = wid + NW * k
        return tbl_hbm.at[f, pl.ds(0, D), pl.ds(vt * 128, 128)]

    def _fire(f, k, par):
        @pl.when(wid + NW * k < VTILES)
        def _():
            pltpu.async_copy(_slab(f, k), stgs[par], sems[par])

    def field_body(f, carry):
        _fire(f, 0, 0)
        _fire(f, 1, 1)

        def kk_body(kk, carry2):
            for par in range(2):
                k = kk * 2 + par
                vt = wid + NW * k

                @pl.when(vt < VTILES)
                def _():
                    pltpu.make_async_copy(
                        _slab(f, k), stgs[par], sems[par]).wait()
                    for d in range(D):
                        i1 = jnp.full((16,), d, jnp.int32)
                        for mb in range(8):
                            val = plsc.load_gather(
                                stgs[par], [i1, midxs[mb]])
                            plsc.store_scatter(out_v, [sidxs[mb] + d], val)
                    pltpu.sync_copy(
                        out_v,
                        out_hbm.at[pl.ds((f * VMAIN + vt * 128) * D, 128 * D)])

                _fire(f, k + 2, par)
            return carry2

        lax.fori_loop(0, 13, kk_body, 0)
        return carry

    lax.fori_loop(0, FSC, field_body, 0)


@functools.partial(
    pl.kernel,
    out_type=jax.ShapeDtypeStruct((A_ROWS * D,), jnp.float32),
    mesh=plsc.VectorSubcoreMesh(core_axis_name="c", subcore_axis_name="s"),
    compiler_params=pltpu.CompilerParams(
        needs_layout_passes=False, use_tc_tiling_on_sc=True),
    scratch_types=[
        pltpu.VMEM((D, 128), jnp.float32),
        pltpu.VMEM((D, 128), jnp.float32),
        pltpu.VMEM((128 * D,), jnp.float32),
        pltpu.SemaphoreType.DMA,
        pltpu.SemaphoreType.DMA,
    ],
)
def _relay_kernel(*refs):
    _relay_body(*refs)


def _relayout_body(in_ref, out_ref):
    x = in_ref[0]
    t1 = x.T
    for q in range(4):
        out_ref[0, :, pl.ds(D * q, D)] = t1[VCH // 4 * q:VCH // 4 * (q + 1), :]


def _relayout_table(tbl_t):
    return pl.pallas_call(
        _relayout_body,
        grid=(FTC, NBLK),
        in_specs=[pl.BlockSpec((1, D, VCH), lambda f, j: (f + FSC, 0, j))],
        out_specs=pl.BlockSpec((1, VCH // 4, 128), lambda f, j: (f, j, 0)),
        out_shape=jax.ShapeDtypeStruct((FTC, ROWS_PER_F, 128), jnp.float32),
    )(tbl_t)


def _post_body(in_ref, out_ref):
    y3 = in_ref[...].reshape(128, NT * D // 128, 128)
    parts = [
        y3[:, r, :].T.reshape(4, D, 128) for r in range(NT * D // 128)
    ]
    out_ref[...] = jnp.concatenate(parts, axis=0)


def _post_transpose(flat128):
    return pl.pallas_call(
        _post_body,
        grid=(B // 128,),
        in_specs=[pl.BlockSpec((NT * D // 128 * 128, 128), lambda j: (j, 0))],
        out_specs=pl.BlockSpec((NT, D, 128), lambda j: (0, 0, j)),
        out_shape=jax.ShapeDtypeStruct((NT, D, B), jnp.float32),
    )(flat128)


def _tok_body(xnum_hbm, idxa_hbm, idxb_hbm, cdsta_hbm, cdstb_hbm, hdst_hbm,
              w_hbm, b_hbm, cls_hbm, tbla_hbm, tblb_hbm, out_hbm,
              idxa_v, idxb_v, cdsta_v, cdstb_v, hdst_v, xnum_v,
              cata_v, catb_v, head_v, w_v, b_v, cls_v, gsem, ssem):
    wid = lax.axis_index("s") * 2 + lax.axis_index("c")
    base = wid * ROWS_PER_W

    pltpu.sync_copy(w_hbm, w_v)
    pltpu.sync_copy(b_hbm, b_v)
    pltpu.sync_copy(cls_hbm, cls_v)
    cls0 = cls_v[pl.ds(0, 16)]
    cls1 = cls_v[pl.ds(16, 16)]

    def chunk_body(c, carry):
        row0 = base + c * NB
        pltpu.sync_copy(idxa_hbm.at[pl.ds(row0 * FSC, NB * FSC)], idxa_v)
        pltpu.sync_copy(idxb_hbm.at[pl.ds(row0 * FTC, NB * FTC)], idxb_v)
        pltpu.sync_copy(cdsta_hbm.at[pl.ds(row0 * FSC, NB * FSC)], cdsta_v)
        pltpu.sync_copy(cdstb_hbm.at[pl.ds(row0 * FTC, NB * FTC)], cdstb_v)
        pltpu.sync_copy(hdst_hbm.at[pl.ds(row0 * (1 + NUM), NB * (1 + NUM))],
                        hdst_v)
        pltpu.sync_copy(xnum_hbm.at[pl.ds(row0 * NUM, NB * NUM)], xnum_v)
        ga = pltpu.async_copy(tbla_hbm.at[idxa_v], cata_v, gsem)
        gb = pltpu.async_copy(tblb_hbm.at[idxb_v], catb_v, gsem)

        def row_body(i, carry2):
            head_v[i * (1 + NUM), pl.ds(0, 16)] = cls0
            head_v[i * (1 + NUM), pl.ds(16, 16)] = cls1
            for j in range(NUM):
                xij = plsc.load_gather(
                    xnum_v, [jnp.full((16,), i * NUM + j, jnp.int32)])
                for h in range(2):
                    off = (2 * j + h) * 16
                    head_v[i * (1 + NUM) + 1 + j, pl.ds(h * 16, 16)] = (
                        xij * w_v[pl.ds(off, 16)] + b_v[pl.ds(off, 16)])
            return carry2

        lax.fori_loop(0, NB, row_body, 0)
        ga.wait()
        gb.wait()
        s1 = pltpu.async_copy(cata_v, out_hbm.at[cdsta_v], ssem)
        s2 = pltpu.async_copy(catb_v, out_hbm.at[cdstb_v], ssem)
        s3 = pltpu.async_copy(head_v, out_hbm.at[hdst_v], ssem)
        s1.wait()
        s2.wait()
        s3.wait()
        return carry

    lax.fori_loop(0, NCHUNKS, chunk_body, 0)


@functools.partial(
    pl.kernel,
    out_type=jax.ShapeDtypeStruct((B * NT, D), jnp.float32),
    mesh=plsc.VectorSubcoreMesh(core_axis_name="c", subcore_axis_name="s"),
    compiler_params=pltpu.CompilerParams(
        needs_layout_passes=False, use_tc_tiling_on_sc=False),
    scratch_types=[
        pltpu.VMEM((NB * FSC,), jnp.int32),
        pltpu.VMEM((NB * FTC,), jnp.int32),
        pltpu.VMEM((NB * FSC,), jnp.int32),
        pltpu.VMEM((NB * FTC,), jnp.int32),
        pltpu.VMEM((NB * (1 + NUM),), jnp.int32),
        pltpu.VMEM((NB * NUM,), jnp.float32),
        pltpu.VMEM((NB * FSC, D), jnp.float32),
        pltpu.VMEM((NB * FTC, D), jnp.float32),
        pltpu.VMEM((NB * (1 + NUM), D), jnp.float32),
        pltpu.VMEM((NUM * D,), jnp.float32),
        pltpu.VMEM((NUM * D,), jnp.float32),
        pltpu.VMEM((D,), jnp.float32),
        pltpu.SemaphoreType.DMA,
        pltpu.SemaphoreType.DMA,
    ],
)
def _tok_kernel(*refs):
    _tok_body(*refs)


def kernel(x_num, x_cat, num_weights, num_bias, cat_tables, cls_token):
    tbl_t = jnp.swapaxes(cat_tables, 1, 2)
    aux = cat_tables[:FSC, VMAIN:, :].reshape(-1)
    tbla = _relay_kernel(tbl_t, aux).reshape(A_ROWS, D)
    tblb128 = _relayout_table(tbl_t)
    tblb = tblb128.reshape(FTC * ROWS_PER_F * 4, D)

    fA = jnp.arange(FSC, dtype=jnp.int32)[None, :]
    vA = x_cat[:, :FSC]
    idxa = jnp.where(vA < VMAIN,
                     fA * VMAIN + vA,
                     FSC * VMAIN + fA * (VOCAB - VMAIN) + (vA - VMAIN))

    fB = jnp.arange(FTC, dtype=jnp.int32)[None, :]
    vB = x_cat[:, FSC:]
    wB = vB & (VCH - 1)
    idxb = (fB * (4 * ROWS_PER_F) + (vB >> 12) * VCH
            + (wB & (VCH // 4 - 1)) * 4 + (wB >> 10))

    brow = jnp.arange(B, dtype=jnp.int32)[:, None] * NT
    cdsta = brow + (1 + NUM) + jnp.arange(FSC, dtype=jnp.int32)[None, :]
    cdstb = brow + (1 + NUM) + FSC + jnp.arange(FTC, dtype=jnp.int32)[None, :]
    hdst = brow + jnp.arange(1 + NUM, dtype=jnp.int32)[None, :]

    flat = _tok_kernel(
        x_num.reshape(-1),
        idxa.reshape(-1),
        idxb.reshape(-1),
        cdsta.reshape(-1),
        cdstb.reshape(-1),
        hdst.reshape(-1),
        num_weights.reshape(-1),
        num_bias.reshape(-1),
        cls_token.reshape(-1),
        tbla,
        tblb,
    )
    out4 = _post_transpose(flat.reshape(B * NT * D // 128, 128))
    return jnp.transpose(out4, (2, 0, 1))


# speedup vs baseline: 1.1138x; 1.1138x over previous
import functools

import jax
import jax.numpy as jnp
from jax import lax
from jax.experimental import pallas as pl
from jax.experimental.pallas import tpu as pltpu
from jax.experimental.pallas import tpu_sc as plsc

B = 16384
NUM = 13
NCAT = 26
VOCAB = 100000
D = 32
NT = 1 + NUM + NCAT
NW = 32
NB = 64
ROWS_PER_W = B // NW
NCHUNKS = ROWS_PER_W // NB

FSC = 9
FTC = NCAT - FSC
VTILES = VOCAB // 128
VMAIN = VTILES * 128
A_ROWS = FSC * VOCAB

VCH = 4096
NBLK = (VOCAB + VCH - 1) // VCH
ROWS_PER_F = NBLK * VCH // 4


def _relay_body(tbl_hbm, aux_hbm, out_hbm, stg0, stg1, out_v, dsem0, dsem1):
    stgs = (stg0, stg1)
    wid = lax.axis_index("s") * 2 + lax.axis_index("c")
    iota16 = lax.iota(jnp.int32, 16)
    midxs = [mb * 16 + iota16 for mb in range(8)]
    sidxs = [m * 32 for m in midxs]
    sems = (dsem0, dsem1)

    @pl.when(wid == 0)
    def _():
        pltpu.sync_copy(aux_hbm, out_hbm.at[pl.ds(FSC * VMAIN * D,
                                                  FSC * (VOCAB - VMAIN) * D)])

    def _slab(f, k):
        vt = wid + NW * k
        return tbl_hbm.at[f, pl.ds(0, D), pl.ds(vt * 128, 128)]

    def _fire(f, k, par):
        @pl.when(wid + NW * k < VTILES)
        def _():
            pltpu.async_copy(_slab(f, k), stgs[par], sems[par])

    def field_body(f, carry):
        _fire(f, 0, 0)
        _fire(f, 1, 1)

        def kk_body(kk, carry2):
            for par in range(2):
                k = kk * 2 + par
                vt = wid + NW * k

                @pl.when(vt < VTILES)
                def _():
                    pltpu.make_async_copy(
                        _slab(f, k), stgs[par], sems[par]).wait()
                    for d in range(D):
                        i1 = jnp.full((16,), d, jnp.int32)
                        for mb in range(8):
                            val = plsc.load_gather(
                                stgs[par], [i1, midxs[mb]])
                            plsc.store_scatter(out_v, [sidxs[mb] + d], val)
                    pltpu.sync_copy(
                        out_v,
                        out_hbm.at[pl.ds((f * VMAIN + vt * 128) * D, 128 * D)])

                _fire(f, k + 2, par)
            return carry2

        lax.fori_loop(0, 13, kk_body, 0)
        return carry

    lax.fori_loop(0, FSC, field_body, 0)


@functools.partial(
    pl.kernel,
    out_type=jax.ShapeDtypeStruct((A_ROWS * D,), jnp.float32),
    mesh=plsc.VectorSubcoreMesh(core_axis_name="c", subcore_axis_name="s"),
    compiler_params=pltpu.CompilerParams(
        needs_layout_passes=False, use_tc_tiling_on_sc=True),
    scratch_types=[
        pltpu.VMEM((D, 128), jnp.float32),
        pltpu.VMEM((D, 128), jnp.float32),
        pltpu.VMEM((128 * D,), jnp.float32),
        pltpu.SemaphoreType.DMA,
        pltpu.SemaphoreType.DMA,
    ],
)
def _relay_kernel(*refs):
    _relay_body(*refs)


def _relayout_body(in_ref, out_ref):
    x = in_ref[0]
    t1 = x.T
    for q in range(4):
        out_ref[0, :, pl.ds(D * q, D)] = t1[VCH // 4 * q:VCH // 4 * (q + 1), :]


def _relayout_table(tbl_t):
    return pl.pallas_call(
        _relayout_body,
        grid=(FTC, NBLK),
        in_specs=[pl.BlockSpec((1, D, VCH), lambda f, j: (f + FSC, 0, j))],
        out_specs=pl.BlockSpec((1, VCH // 4, 128), lambda f, j: (f, j, 0)),
        out_shape=jax.ShapeDtypeStruct((FTC, ROWS_PER_F, 128), jnp.float32),
    )(tbl_t)


PB = 2


def _post_body(in_ref, out_ref):
    for t in range(PB):
        y3 = in_ref[pl.ds(t * NT * D, NT * D), :].reshape(
            128, NT * D // 128, 128)
        parts = [
            y3[:, r, :].T.reshape(4, D, 128) for r in range(NT * D // 128)
        ]
        out_ref[:, :, pl.ds(t * 128, 128)] = jnp.concatenate(parts, axis=0)


def _post_transpose(flat128):
    return pl.pallas_call(
        _post_body,
        grid=(B // (128 * PB),),
        in_specs=[pl.BlockSpec((PB * NT * D, 128), lambda j: (j, 0))],
        out_specs=pl.BlockSpec((NT, D, 128 * PB), lambda j: (0, 0, j)),
        out_shape=jax.ShapeDtypeStruct((NT, D, B), jnp.float32),
    )(flat128)


def _tok_body(xnum_hbm, idxa_hbm, idxb_hbm, cdsta_hbm, cdstb_hbm, hdst_hbm,
              w_hbm, b_hbm, cls_hbm, tbla_hbm, tblb_hbm, out_hbm,
              idxa_v, idxb_v, cdsta_v, cdstb_v, hdst_v, xnum_v,
              cata_v, catb_v, head_v, w_v, b_v, cls_v, gsem, ssem):
    wid = lax.axis_index("s") * 2 + lax.axis_index("c")
    base = wid * ROWS_PER_W

    pltpu.sync_copy(w_hbm, w_v)
    pltpu.sync_copy(b_hbm, b_v)
    pltpu.sync_copy(cls_hbm, cls_v)
    cls0 = cls_v[pl.ds(0, 16)]
    cls1 = cls_v[pl.ds(16, 16)]

    def chunk_body(c, carry):
        row0 = base + c * NB
        pltpu.sync_copy(idxa_hbm.at[pl.ds(row0 * FSC, NB * FSC)], idxa_v)
        pltpu.sync_copy(idxb_hbm.at[pl.ds(row0 * FTC, NB * FTC)], idxb_v)
        pltpu.sync_copy(cdsta_hbm.at[pl.ds(row0 * FSC, NB * FSC)], cdsta_v)
        pltpu.sync_copy(cdstb_hbm.at[pl.ds(row0 * FTC, NB * FTC)], cdstb_v)
        pltpu.sync_copy(hdst_hbm.at[pl.ds(row0 * (1 + NUM), NB * (1 + NUM))],
                        hdst_v)
        pltpu.sync_copy(xnum_hbm.at[pl.ds(row0 * NUM, NB * NUM)], xnum_v)
        ga = pltpu.async_copy(tbla_hbm.at[idxa_v], cata_v, gsem)
        gb = pltpu.async_copy(tblb_hbm.at[idxb_v], catb_v, gsem)

        def row_body(i, carry2):
            head_v[i * (1 + NUM), pl.ds(0, 16)] = cls0
            head_v[i * (1 + NUM), pl.ds(16, 16)] = cls1
            for j in range(NUM):
                xij = plsc.load_gather(
                    xnum_v, [jnp.full((16,), i * NUM + j, jnp.int32)])
                for h in range(2):
                    off = (2 * j + h) * 16
                    head_v[i * (1 + NUM) + 1 + j, pl.ds(h * 16, 16)] = (
                        xij * w_v[pl.ds(off, 16)] + b_v[pl.ds(off, 16)])
            return carry2

        lax.fori_loop(0, NB, row_body, 0)
        ga.wait()
        gb.wait()
        s1 = pltpu.async_copy(cata_v, out_hbm.at[cdsta_v], ssem)
        s2 = pltpu.async_copy(catb_v, out_hbm.at[cdstb_v], ssem)
        s3 = pltpu.async_copy(head_v, out_hbm.at[hdst_v], ssem)
        s1.wait()
        s2.wait()
        s3.wait()
        return carry

    lax.fori_loop(0, NCHUNKS, chunk_body, 0)


@functools.partial(
    pl.kernel,
    out_type=jax.ShapeDtypeStruct((B * NT, D), jnp.float32),
    mesh=plsc.VectorSubcoreMesh(core_axis_name="c", subcore_axis_name="s"),
    compiler_params=pltpu.CompilerParams(
        needs_layout_passes=False, use_tc_tiling_on_sc=False),
    scratch_types=[
        pltpu.VMEM((NB * FSC,), jnp.int32),
        pltpu.VMEM((NB * FTC,), jnp.int32),
        pltpu.VMEM((NB * FSC,), jnp.int32),
        pltpu.VMEM((NB * FTC,), jnp.int32),
        pltpu.VMEM((NB * (1 + NUM),), jnp.int32),
        pltpu.VMEM((NB * NUM,), jnp.float32),
        pltpu.VMEM((NB * FSC, D), jnp.float32),
        pltpu.VMEM((NB * FTC, D), jnp.float32),
        pltpu.VMEM((NB * (1 + NUM), D), jnp.float32),
        pltpu.VMEM((NUM * D,), jnp.float32),
        pltpu.VMEM((NUM * D,), jnp.float32),
        pltpu.VMEM((D,), jnp.float32),
        pltpu.SemaphoreType.DMA,
        pltpu.SemaphoreType.DMA,
    ],
)
def _tok_kernel(*refs):
    _tok_body(*refs)


def kernel(x_num, x_cat, num_weights, num_bias, cat_tables, cls_token):
    tbl_t = jnp.swapaxes(cat_tables, 1, 2)
    aux = cat_tables[:FSC, VMAIN:, :].reshape(-1)
    tbla = _relay_kernel(tbl_t, aux).reshape(A_ROWS, D)
    tblb128 = _relayout_table(tbl_t)
    tblb = tblb128.reshape(FTC * ROWS_PER_F * 4, D)

    fA = jnp.arange(FSC, dtype=jnp.int32)[None, :]
    vA = x_cat[:, :FSC]
    idxa = jnp.where(vA < VMAIN,
                     fA * VMAIN + vA,
                     FSC * VMAIN + fA * (VOCAB - VMAIN) + (vA - VMAIN))

    fB = jnp.arange(FTC, dtype=jnp.int32)[None, :]
    vB = x_cat[:, FSC:]
    wB = vB & (VCH - 1)
    idxb = (fB * (4 * ROWS_PER_F) + (vB >> 12) * VCH
            + (wB & (VCH // 4 - 1)) * 4 + (wB >> 10))

    brow = jnp.arange(B, dtype=jnp.int32)[:, None] * NT
    cdsta = brow + (1 + NUM) + jnp.arange(FSC, dtype=jnp.int32)[None, :]
    cdstb = brow + (1 + NUM) + FSC + jnp.arange(FTC, dtype=jnp.int32)[None, :]
    hdst = brow + jnp.arange(1 + NUM, dtype=jnp.int32)[None, :]

    flat = _tok_kernel(
        x_num.reshape(-1),
        idxa.reshape(-1),
        idxb.reshape(-1),
        cdsta.reshape(-1),
        cdstb.reshape(-1),
        hdst.reshape(-1),
        num_weights.reshape(-1),
        num_bias.reshape(-1),
        cls_token.reshape(-1),
        tbla,
        tblb,
    )
    out4 = _post_transpose(flat.reshape(B * NT * D // 128, 128))
    return jnp.transpose(out4, (2, 0, 1))


# speedup vs baseline: 1.1381x; 1.0218x over previous
import functools

import jax
import jax.numpy as jnp
from jax import lax
from jax.experimental import pallas as pl
from jax.experimental.pallas import tpu as pltpu
from jax.experimental.pallas import tpu_sc as plsc

B = 16384
NUM = 13
NCAT = 26
VOCAB = 100000
D = 32
NT = 1 + NUM + NCAT
NW = 32
NB = 64
ROWS_PER_W = B // NW
NCHUNKS = ROWS_PER_W // NB

FSC = 9
FTC = NCAT - FSC
VTILES = VOCAB // 128
VMAIN = VTILES * 128
A_ROWS = FSC * VOCAB

VCH = 4096
NBLK = (VOCAB + VCH - 1) // VCH
ROWS_PER_F = NBLK * VCH // 4


def _relay_body(tbl_hbm, aux_hbm, out_hbm, stg0, stg1, out_v, dsem0, dsem1):
    stgs = (stg0, stg1)
    wid = lax.axis_index("s") * 2 + lax.axis_index("c")
    iota16 = lax.iota(jnp.int32, 16)
    midxs = [mb * 16 + iota16 for mb in range(8)]
    sidxs = [m * 32 for m in midxs]
    sems = (dsem0, dsem1)

    @pl.when(wid == 0)
    def _():
        pltpu.sync_copy(aux_hbm, out_hbm.at[pl.ds(FSC * VMAIN * D,
                                                  FSC * (VOCAB - VMAIN) * D)])

    def _slab(f, k):
        vt = wid + NW * k
        return tbl_hbm.at[f, pl.ds(0, D), pl.ds(vt * 128, 128)]

    def _fire(f, k, par):
        @pl.when(wid + NW * k < VTILES)
        def _():
            pltpu.async_copy(_slab(f, k), stgs[par], sems[par])

    def field_body(f, carry):
        _fire(f, 0, 0)
        _fire(f, 1, 1)

        def kk_body(kk, carry2):
            for par in range(2):
                k = kk * 2 + par
                vt = wid + NW * k

                @pl.when(vt < VTILES)
                def _():
                    pltpu.make_async_copy(
                        _slab(f, k), stgs[par], sems[par]).wait()
                    for d in range(D):
                        i1 = jnp.full((16,), d, jnp.int32)
                        for mb in range(8):
                            val = plsc.load_gather(
                                stgs[par], [i1, midxs[mb]])
                            plsc.store_scatter(out_v, [sidxs[mb] + d], val)
                    pltpu.sync_copy(
                        out_v,
                        out_hbm.at[pl.ds((f * VMAIN + vt * 128) * D, 128 * D)])

                _fire(f, k + 2, par)
            return carry2

        lax.fori_loop(0, 13, kk_body, 0)
        return carry

    lax.fori_loop(0, FSC, field_body, 0)


@functools.partial(
    pl.kernel,
    out_type=jax.ShapeDtypeStruct((A_ROWS * D,), jnp.float32),
    mesh=plsc.VectorSubcoreMesh(core_axis_name="c", subcore_axis_name="s"),
    compiler_params=pltpu.CompilerParams(
        needs_layout_passes=False, use_tc_tiling_on_sc=True),
    scratch_types=[
        pltpu.VMEM((D, 128), jnp.float32),
        pltpu.VMEM((D, 128), jnp.float32),
        pltpu.VMEM((128 * D,), jnp.float32),
        pltpu.SemaphoreType.DMA,
        pltpu.SemaphoreType.DMA,
    ],
)
def _relay_kernel(*refs):
    _relay_body(*refs)


def _relayout_body(in_ref, out_ref):
    x = in_ref[0]
    t1 = x.T
    for q in range(4):
        out_ref[0, :, pl.ds(D * q, D)] = t1[VCH // 4 * q:VCH // 4 * (q + 1), :]


def _relayout_table(tbl_t):
    return pl.pallas_call(
        _relayout_body,
        grid=(FTC, NBLK),
        in_specs=[pl.BlockSpec((1, D, VCH), lambda f, j: (f + FSC, 0, j))],
        out_specs=pl.BlockSpec((1, VCH // 4, 128), lambda f, j: (f, j, 0)),
        out_shape=jax.ShapeDtypeStruct((FTC, ROWS_PER_F, 128), jnp.float32),
    )(tbl_t)


PB = 4


def _post_body(in_ref, out_ref):
    for t in range(PB):
        y3 = in_ref[pl.ds(t * NT * D, NT * D), :].reshape(
            128, NT * D // 128, 128)
        parts = [
            y3[:, r, :].T.reshape(4, D, 128) for r in range(NT * D // 128)
        ]
        out_ref[:, :, pl.ds(t * 128, 128)] = jnp.concatenate(parts, axis=0)


def _post_transpose(flat128):
    return pl.pallas_call(
        _post_body,
        grid=(B // (128 * PB),),
        in_specs=[pl.BlockSpec((PB * NT * D, 128), lambda j: (j, 0))],
        out_specs=pl.BlockSpec((NT, D, 128 * PB), lambda j: (0, 0, j)),
        out_shape=jax.ShapeDtypeStruct((NT, D, B), jnp.float32),
    )(flat128)


def _tok_body(xnum_hbm, idxa_hbm, idxb_hbm, cdsta_hbm, cdstb_hbm, hdst_hbm,
              w_hbm, b_hbm, cls_hbm, tbla_hbm, tblb_hbm, out_hbm,
              idxa_v, idxb_v, cdsta_v, cdstb_v, hdst_v, xnum_v,
              cata_v, catb_v, head_v, w_v, b_v, cls_v, gsem, ssem):
    wid = lax.axis_index("s") * 2 + lax.axis_index("c")
    base = wid * ROWS_PER_W

    pltpu.sync_copy(w_hbm, w_v)
    pltpu.sync_copy(b_hbm, b_v)
    pltpu.sync_copy(cls_hbm, cls_v)
    cls0 = cls_v[pl.ds(0, 16)]
    cls1 = cls_v[pl.ds(16, 16)]

    def chunk_body(c, carry):
        row0 = base + c * NB
        pltpu.sync_copy(idxa_hbm.at[pl.ds(row0 * FSC, NB * FSC)], idxa_v)
        pltpu.sync_copy(idxb_hbm.at[pl.ds(row0 * FTC, NB * FTC)], idxb_v)
        pltpu.sync_copy(cdsta_hbm.at[pl.ds(row0 * FSC, NB * FSC)], cdsta_v)
        pltpu.sync_copy(cdstb_hbm.at[pl.ds(row0 * FTC, NB * FTC)], cdstb_v)
        pltpu.sync_copy(hdst_hbm.at[pl.ds(row0 * (1 + NUM), NB * (1 + NUM))],
                        hdst_v)
        pltpu.sync_copy(xnum_hbm.at[pl.ds(row0 * NUM, NB * NUM)], xnum_v)
        ga = pltpu.async_copy(tbla_hbm.at[idxa_v], cata_v, gsem)
        gb = pltpu.async_copy(tblb_hbm.at[idxb_v], catb_v, gsem)

        def row_body(i, carry2):
            head_v[i * (1 + NUM), pl.ds(0, 16)] = cls0
            head_v[i * (1 + NUM), pl.ds(16, 16)] = cls1
            for j in range(NUM):
                xij = plsc.load_gather(
                    xnum_v, [jnp.full((16,), i * NUM + j, jnp.int32)])
                for h in range(2):
                    off = (2 * j + h) * 16
                    head_v[i * (1 + NUM) + 1 + j, pl.ds(h * 16, 16)] = (
                        xij * w_v[pl.ds(off, 16)] + b_v[pl.ds(off, 16)])
            return carry2

        lax.fori_loop(0, NB, row_body, 0)
        ga.wait()
        gb.wait()
        s1 = pltpu.async_copy(cata_v, out_hbm.at[cdsta_v], ssem)
        s2 = pltpu.async_copy(catb_v, out_hbm.at[cdstb_v], ssem)
        s3 = pltpu.async_copy(head_v, out_hbm.at[hdst_v], ssem)
        s1.wait()
        s2.wait()
        s3.wait()
        return carry

    lax.fori_loop(0, NCHUNKS, chunk_body, 0)


@functools.partial(
    pl.kernel,
    out_type=jax.ShapeDtypeStruct((B * NT, D), jnp.float32),
    mesh=plsc.VectorSubcoreMesh(core_axis_name="c", subcore_axis_name="s"),
    compiler_params=pltpu.CompilerParams(
        needs_layout_passes=False, use_tc_tiling_on_sc=False),
    scratch_types=[
        pltpu.VMEM((NB * FSC,), jnp.int32),
        pltpu.VMEM((NB * FTC,), jnp.int32),
        pltpu.VMEM((NB * FSC,), jnp.int32),
        pltpu.VMEM((NB * FTC,), jnp.int32),
        pltpu.VMEM((NB * (1 + NUM),), jnp.int32),
        pltpu.VMEM((NB * NUM,), jnp.float32),
        pltpu.VMEM((NB * FSC, D), jnp.float32),
        pltpu.VMEM((NB * FTC, D), jnp.float32),
        pltpu.VMEM((NB * (1 + NUM), D), jnp.float32),
        pltpu.VMEM((NUM * D,), jnp.float32),
        pltpu.VMEM((NUM * D,), jnp.float32),
        pltpu.VMEM((D,), jnp.float32),
        pltpu.SemaphoreType.DMA,
        pltpu.SemaphoreType.DMA,
    ],
)
def _tok_kernel(*refs):
    _tok_body(*refs)


def kernel(x_num, x_cat, num_weights, num_bias, cat_tables, cls_token):
    tbl_t = jnp.swapaxes(cat_tables, 1, 2)
    aux = cat_tables[:FSC, VMAIN:, :].reshape(-1)
    tbla = _relay_kernel(tbl_t, aux).reshape(A_ROWS, D)
    tblb128 = _relayout_table(tbl_t)
    tblb = tblb128.reshape(FTC * ROWS_PER_F * 4, D)

    fA = jnp.arange(FSC, dtype=jnp.int32)[None, :]
    vA = x_cat[:, :FSC]
    idxa = jnp.where(vA < VMAIN,
                     fA * VMAIN + vA,
                     FSC * VMAIN + fA * (VOCAB - VMAIN) + (vA - VMAIN))

    fB = jnp.arange(FTC, dtype=jnp.int32)[None, :]
    vB = x_cat[:, FSC:]
    wB = vB & (VCH - 1)
    idxb = (fB * (4 * ROWS_PER_F) + (vB >> 12) * VCH
            + (wB & (VCH // 4 - 1)) * 4 + (wB >> 10))

    brow = jnp.arange(B, dtype=jnp.int32)[:, None] * NT
    cdsta = brow + (1 + NUM) + jnp.arange(FSC, dtype=jnp.int32)[None, :]
    cdstb = brow + (1 + NUM) + FSC + jnp.arange(FTC, dtype=jnp.int32)[None, :]
    hdst = brow + jnp.arange(1 + NUM, dtype=jnp.int32)[None, :]

    flat = _tok_kernel(
        x_num.reshape(-1),
        idxa.reshape(-1),
        idxb.reshape(-1),
        cdsta.reshape(-1),
        cdstb.reshape(-1),
        hdst.reshape(-1),
        num_weights.reshape(-1),
        num_bias.reshape(-1),
        cls_token.reshape(-1),
        tbla,
        tblb,
    )
    out4 = _post_transpose(flat.reshape(B * NT * D // 128, 128))
    return jnp.transpose(out4, (2, 0, 1))


# speedup vs baseline: 1.1665x; 1.0249x over previous
import functools

import jax
import jax.numpy as jnp
from jax import lax
from jax.experimental import pallas as pl
from jax.experimental.pallas import tpu as pltpu
from jax.experimental.pallas import tpu_sc as plsc

B = 16384
NUM = 13
NCAT = 26
VOCAB = 100000
D = 32
NT = 1 + NUM + NCAT
NW = 32
NB = 64
ROWS_PER_W = B // NW
NCHUNKS = ROWS_PER_W // NB

FSC = 9
FTC = NCAT - FSC
VTILES = VOCAB // 128
VMAIN = VTILES * 128
A_ROWS = FSC * VOCAB

VCH = 4096
NBLK = (VOCAB + VCH - 1) // VCH
ROWS_PER_F = NBLK * VCH // 4


def _relay_body(tbl_hbm, aux_hbm, out_hbm, stg0, stg1, out_v, dsem0, dsem1):
    stgs = (stg0, stg1)
    wid = lax.axis_index("s") * 2 + lax.axis_index("c")
    iota16 = lax.iota(jnp.int32, 16)
    midxs = [mb * 16 + iota16 for mb in range(8)]
    sidxs = [m * 32 for m in midxs]
    sems = (dsem0, dsem1)

    @pl.when(wid == 0)
    def _():
        pltpu.sync_copy(aux_hbm, out_hbm.at[pl.ds(FSC * VMAIN * D,
                                                  FSC * (VOCAB - VMAIN) * D)])

    def _slab(f, k):
        vt = wid + NW * k
        return tbl_hbm.at[f, pl.ds(0, D), pl.ds(vt * 128, 128)]

    def _fire(f, k, par):
        @pl.when(wid + NW * k < VTILES)
        def _():
            pltpu.async_copy(_slab(f, k), stgs[par], sems[par])

    def field_body(f, carry):
        _fire(f, 0, 0)
        _fire(f, 1, 1)

        def kk_body(kk, carry2):
            for par in range(2):
                k = kk * 2 + par
                vt = wid + NW * k

                @pl.when(vt < VTILES)
                def _():
                    pltpu.make_async_copy(
                        _slab(f, k), stgs[par], sems[par]).wait()
                    for d in range(D):
                        i1 = jnp.full((16,), d, jnp.int32)
                        for mb in range(8):
                            val = plsc.load_gather(
                                stgs[par], [i1, midxs[mb]])
                            plsc.store_scatter(out_v, [sidxs[mb] + d], val)
                    pltpu.sync_copy(
                        out_v,
                        out_hbm.at[pl.ds((f * VMAIN + vt * 128) * D, 128 * D)])

                _fire(f, k + 2, par)
            return carry2

        lax.fori_loop(0, 13, kk_body, 0)
        return carry

    lax.fori_loop(0, FSC, field_body, 0)


@functools.partial(
    pl.kernel,
    out_type=jax.ShapeDtypeStruct((A_ROWS * D,), jnp.float32),
    mesh=plsc.VectorSubcoreMesh(core_axis_name="c", subcore_axis_name="s"),
    compiler_params=pltpu.CompilerParams(
        needs_layout_passes=False, use_tc_tiling_on_sc=True),
    scratch_types=[
        pltpu.VMEM((D, 128), jnp.float32),
        pltpu.VMEM((D, 128), jnp.float32),
        pltpu.VMEM((128 * D,), jnp.float32),
        pltpu.SemaphoreType.DMA,
        pltpu.SemaphoreType.DMA,
    ],
)
def _relay_kernel(*refs):
    _relay_body(*refs)


def _relayout_body(in_ref, out_ref):
    x = in_ref[0]
    t1 = x.T
    for q in range(4):
        out_ref[0, :, pl.ds(D * q, D)] = t1[VCH // 4 * q:VCH // 4 * (q + 1), :]


def _relayout_table(tbl_t):
    return pl.pallas_call(
        _relayout_body,
        grid=(FTC, NBLK),
        in_specs=[pl.BlockSpec((1, D, VCH), lambda f, j: (f + FSC, 0, j))],
        out_specs=pl.BlockSpec((1, VCH // 4, 128), lambda f, j: (f, j, 0)),
        out_shape=jax.ShapeDtypeStruct((FTC, ROWS_PER_F, 128), jnp.float32),
    )(tbl_t)


PB = 4


def _post_body(in_ref, out_ref):
    for t in range(PB):
        y3 = in_ref[pl.ds(t * NT * D, NT * D), :].reshape(
            128, NT * D // 128, 128)
        parts = [
            y3[:, r, :].T.reshape(4, D, 128) for r in range(NT * D // 128)
        ]
        out_ref[:, :, pl.ds(t * 128, 128)] = jnp.concatenate(parts, axis=0)


def _post_transpose(flat128):
    return pl.pallas_call(
        _post_body,
        grid=(B // (128 * PB),),
        in_specs=[pl.BlockSpec((PB * NT * D, 128), lambda j: (j, 0))],
        out_specs=pl.BlockSpec((NT, D, 128 * PB), lambda j: (0, 0, j)),
        out_shape=jax.ShapeDtypeStruct((NT, D, B), jnp.float32),
    )(flat128)


def _tok_body(xnum_hbm, idxa_hbm, idxb_hbm, cdsta_hbm, cdstb_hbm, hdst_hbm,
              w_hbm, b_hbm, cls_hbm, tbla_hbm, tblb_hbm, out_hbm,
              idxa_v, idxb_v, cdsta_v, cdstb_v, hdst_v, xnum_v,
              cata_v, catb_v, head_v, w_v, b_v, cls_v, gsem, ssem,
              isem1, isem2):
    wid = lax.axis_index("s") * 2 + lax.axis_index("c")
    base = wid * ROWS_PER_W

    pltpu.sync_copy(w_hbm, w_v)
    pltpu.sync_copy(b_hbm, b_v)
    pltpu.sync_copy(cls_hbm, cls_v)
    cls0 = cls_v[pl.ds(0, 16)]
    cls1 = cls_v[pl.ds(16, 16)]

    def chunk_body(c, carry):
        row0 = base + c * NB
        la = pltpu.async_copy(
            idxa_hbm.at[pl.ds(row0 * FSC, NB * FSC)], idxa_v, isem1)
        lb = pltpu.async_copy(
            idxb_hbm.at[pl.ds(row0 * FTC, NB * FTC)], idxb_v, isem1)
        l3 = pltpu.async_copy(
            cdsta_hbm.at[pl.ds(row0 * FSC, NB * FSC)], cdsta_v, isem2)
        l4 = pltpu.async_copy(
            cdstb_hbm.at[pl.ds(row0 * FTC, NB * FTC)], cdstb_v, isem2)
        l5 = pltpu.async_copy(
            hdst_hbm.at[pl.ds(row0 * (1 + NUM), NB * (1 + NUM))], hdst_v,
            isem2)
        l6 = pltpu.async_copy(
            xnum_hbm.at[pl.ds(row0 * NUM, NB * NUM)], xnum_v, isem2)
        la.wait()
        lb.wait()
        ga = pltpu.async_copy(tbla_hbm.at[idxa_v], cata_v, gsem)
        gb = pltpu.async_copy(tblb_hbm.at[idxb_v], catb_v, gsem)
        l3.wait()
        l4.wait()
        l5.wait()
        l6.wait()

        def row_body(i, carry2):
            head_v[i * (1 + NUM), pl.ds(0, 16)] = cls0
            head_v[i * (1 + NUM), pl.ds(16, 16)] = cls1
            for j in range(NUM):
                xij = plsc.load_gather(
                    xnum_v, [jnp.full((16,), i * NUM + j, jnp.int32)])
                for h in range(2):
                    off = (2 * j + h) * 16
                    head_v[i * (1 + NUM) + 1 + j, pl.ds(h * 16, 16)] = (
                        xij * w_v[pl.ds(off, 16)] + b_v[pl.ds(off, 16)])
            return carry2

        lax.fori_loop(0, NB, row_body, 0)
        ga.wait()
        gb.wait()
        s1 = pltpu.async_copy(cata_v, out_hbm.at[cdsta_v], ssem)
        s2 = pltpu.async_copy(catb_v, out_hbm.at[cdstb_v], ssem)
        s3 = pltpu.async_copy(head_v, out_hbm.at[hdst_v], ssem)
        s1.wait()
        s2.wait()
        s3.wait()
        return carry

    lax.fori_loop(0, NCHUNKS, chunk_body, 0)


@functools.partial(
    pl.kernel,
    out_type=jax.ShapeDtypeStruct((B * NT, D), jnp.float32),
    mesh=plsc.VectorSubcoreMesh(core_axis_name="c", subcore_axis_name="s"),
    compiler_params=pltpu.CompilerParams(
        needs_layout_passes=False, use_tc_tiling_on_sc=False),
    scratch_types=[
        pltpu.VMEM((NB * FSC,), jnp.int32),
        pltpu.VMEM((NB * FTC,), jnp.int32),
        pltpu.VMEM((NB * FSC,), jnp.int32),
        pltpu.VMEM((NB * FTC,), jnp.int32),
        pltpu.VMEM((NB * (1 + NUM),), jnp.int32),
        pltpu.VMEM((NB * NUM,), jnp.float32),
        pltpu.VMEM((NB * FSC, D), jnp.float32),
        pltpu.VMEM((NB * FTC, D), jnp.float32),
        pltpu.VMEM((NB * (1 + NUM), D), jnp.float32),
        pltpu.VMEM((NUM * D,), jnp.float32),
        pltpu.VMEM((NUM * D,), jnp.float32),
        pltpu.VMEM((D,), jnp.float32),
        pltpu.SemaphoreType.DMA,
        pltpu.SemaphoreType.DMA,
        pltpu.SemaphoreType.DMA,
        pltpu.SemaphoreType.DMA,
    ],
)
def _tok_kernel(*refs):
    _tok_body(*refs)


def kernel(x_num, x_cat, num_weights, num_bias, cat_tables, cls_token):
    tbl_t = jnp.swapaxes(cat_tables, 1, 2)
    aux = cat_tables[:FSC, VMAIN:, :].reshape(-1)
    tbla = _relay_kernel(tbl_t, aux).reshape(A_ROWS, D)
    tblb128 = _relayout_table(tbl_t)
    tblb = tblb128.reshape(FTC * ROWS_PER_F * 4, D)

    fA = jnp.arange(FSC, dtype=jnp.int32)[None, :]
    vA = x_cat[:, :FSC]
    idxa = jnp.where(vA < VMAIN,
                     fA * VMAIN + vA,
                     FSC * VMAIN + fA * (VOCAB - VMAIN) + (vA - VMAIN))

    fB = jnp.arange(FTC, dtype=jnp.int32)[None, :]
    vB = x_cat[:, FSC:]
    wB = vB & (VCH - 1)
    idxb = (fB * (4 * ROWS_PER_F) + (vB >> 12) * VCH
            + (wB & (VCH // 4 - 1)) * 4 + (wB >> 10))

    brow = jnp.arange(B, dtype=jnp.int32)[:, None] * NT
    cdsta = brow + (1 + NUM) + jnp.arange(FSC, dtype=jnp.int32)[None, :]
    cdstb = brow + (1 + NUM) + FSC + jnp.arange(FTC, dtype=jnp.int32)[None, :]
    hdst = brow + jnp.arange(1 + NUM, dtype=jnp.int32)[None, :]

    flat = _tok_kernel(
        x_num.reshape(-1),
        idxa.reshape(-1),
        idxb.reshape(-1),
        cdsta.reshape(-1),
        cdstb.reshape(-1),
        hdst.reshape(-1),
        num_weights.reshape(-1),
        num_bias.reshape(-1),
        cls_token.reshape(-1),
        tbla,
        tblb,
    )
    out4 = _post_transpose(flat.reshape(B * NT * D // 128, 128))
    return jnp.transpose(out4, (2, 0, 1))
